# baseline (speedup 1.0000x reference)
import functools

import jax
import jax.numpy as jnp
from jax import lax
from jax.experimental import pallas as pl
from jax.experimental.pallas import tpu as pltpu
from jax.experimental.pallas import tpu_sc as plsc

_NC = 2
_NS = 16
_L = 16
_CHUNK = 112
_NROW = 3
_NIDX = 6


def _matmul(x, W):
    n, d_in = x.shape
    d_out = W.shape[1]
    blk = 1000

    def body(x_ref, w_ref, o_ref):
        o_ref[...] = jnp.dot(x_ref[...], w_ref[...],
                             preferred_element_type=jnp.float32)

    return pl.pallas_call(
        body,
        grid=(n // blk,),
        in_specs=[pl.BlockSpec((blk, d_in), lambda i: (i, 0)),
                  pl.BlockSpec((d_in, d_out), lambda i: (0, 0))],
        out_specs=pl.BlockSpec((blk, d_out), lambda i: (i, 0)),
        out_shape=jax.ShapeDtypeStruct((n, d_out), jnp.float32),
    )(x, W)


def _combine(partials, b):
    _, n, d = partials.shape
    blk = 1000
    b2 = b.reshape(1, d).astype(jnp.float32)

    def body(p_ref, b_ref, o_ref):
        o_ref[...] = p_ref[0] + p_ref[1] + b_ref[...]

    return pl.pallas_call(
        body,
        grid=(n // blk,),
        in_specs=[pl.BlockSpec((2, blk, d), lambda i: (0, i, 0)),
                  pl.BlockSpec((1, d), lambda i: (0, 0))],
        out_specs=pl.BlockSpec((blk, d), lambda i: (i, 0)),
        out_shape=jax.ShapeDtypeStruct((n, d), jnp.float32),
    )(partials, b2)


def _sc_aggregate(support, eidx, ew, k_real):
    n, d = support.shape
    rows_per_tile = -(-(-(-n // _NS)) // 8) * 8
    n_pad = _NS * rows_per_tile
    mesh = plsc.VectorSubcoreMesh(core_axis_name="c", subcore_axis_name="s")

    @functools.partial(
        pl.kernel,
        out_type=jax.ShapeDtypeStruct((_NC, n_pad, d), jnp.float32),
        mesh=mesh,
        scratch_types=(
            [pltpu.VMEM((_CHUNK, d), jnp.float32) for _ in range(_NROW)]
            + [pltpu.VMEM((2, _CHUNK), jnp.int32) for _ in range(_NIDX)]
            + [pltpu.VMEM((_CHUNK,), jnp.float32) for _ in range(_NIDX)]
            + [pltpu.VMEM_SHARED((n_pad, d), jnp.float32)]
            + [pltpu.SemaphoreType.DMA] * (2 * _NROW + _NIDX)
        ),
    )
    def agg(support_hbm, eidx_hbm, ew_hbm, out_hbm, *refs):
        rows = refs[:_NROW]
        slots = refs[_NROW:_NROW + _NIDX]
        wslots = refs[_NROW + _NIDX:_NROW + 2 * _NIDX]
        acc_sh = refs[_NROW + 2 * _NIDX]
        sems = refs[_NROW + 2 * _NIDX + 1:]
        gsems = sems[:_NROW]
        ssems = sems[_NROW:2 * _NROW]
        isems = sems[2 * _NROW:]

        c = lax.axis_index("c")
        s = lax.axis_index("s")
        wid = s * _NC + c

        def zero_row(r, carry):
            for dd in range(d // _L):
                rows[0][r, pl.ds(dd * _L, _L)] = jnp.zeros((_L,), jnp.float32)
            return carry

        lax.fori_loop(0, _CHUNK, zero_row, 0)
        base = s * rows_per_tile
        nfull = rows_per_tile // _CHUNK
        for t in range(nfull):
            pltpu.sync_copy(rows[0],
                            acc_sh.at[pl.ds(base + t * _CHUNK, _CHUNK)])
        rem = rows_per_tile - nfull * _CHUNK
        if rem:
            pltpu.sync_copy(rows[0].at[pl.ds(0, rem)],
                            acc_sh.at[pl.ds(base + nfull * _CHUNK, rem)])
        plsc.subcore_barrier()

        def fire_idx(g, si):
            pltpu.async_copy(eidx_hbm.at[wid, g], slots[si], isems[si])
            pltpu.async_copy(ew_hbm.at[wid, g], wslots[si], isems[si])

        def wait_idx(si):
            pltpu.make_async_copy(eidx_hbm.at[0, 0], slots[si],
                                  isems[si]).wait()
            pltpu.make_async_copy(ew_hbm.at[0, 0], wslots[si],
                                  isems[si]).wait()

        def fire_gather(si, ri):
            pltpu.async_copy(support_hbm.at[slots[si].at[0]], rows[ri],
                             gsems[ri])

        def wait_gather(ri):
            pltpu.make_async_copy(support_hbm.at[pl.ds(0, _CHUNK)], rows[ri],
                                  gsems[ri]).wait()

        def scale(si, ri):
            buf = rows[ri]
            wrow = wslots[si]

            def scale_group(grp, inner):
                wvec = wrow[pl.ds(grp * _L, _L)]
                for rr in range(_L):
                    ws = wvec[rr]
                    r = grp * _L + rr
                    for dd in range(d // _L):
                        sl = pl.ds(dd * _L, _L)
                        buf[r, sl] = buf[r, sl] * ws
                return inner

            lax.fori_loop(0, _CHUNK // _L, scale_group, 0)

        def fire_scatter(si, ri):
            pltpu.async_copy(rows[ri], acc_sh.at[slots[si].at[1]], ssems[ri],
                             add=True)

        def wait_scatter(ri):
            pltpu.make_async_copy(support_hbm.at[pl.ds(0, _CHUNK)], rows[ri],
                                  ssems[ri]).wait()

        for g in range(3):
            fire_idx(g, g)
        wait_idx(0)
        fire_gather(0, 0)
        wait_idx(1)
        fire_gather(1, 1)

        def step(g, ii):
            ri = ii % _NROW
            wait_gather(ri)
            scale(ii, ri)
            fire_scatter(ii, ri)
            ri_next = (ii + 2) % _NROW

            @pl.when(g >= 1)
            def _():
                wait_scatter(ri_next)

            fire_idx(g + 3, (ii + 3) % _NIDX)
            wait_idx((ii + 2) % _NIDX)
            fire_gather((ii + 2) % _NIDX, ri_next)

        def block_body(g0, carry):
            for ii in range(_NIDX):
                step(g0 * _NIDX + ii, ii)
            return carry

        lax.fori_loop(0, k_real // _NIDX, block_body, 0)

        wait_gather(k_real % _NROW)
        wait_gather((k_real + 1) % _NROW)
        wait_idx((k_real + 2) % _NIDX)
        wait_scatter((k_real - 1) % _NROW)
        plsc.subcore_barrier()

        pltpu.sync_copy(acc_sh.at[pl.ds(base, rows_per_tile)],
                        out_hbm.at[c, pl.ds(base, rows_per_tile)])

    return agg(support, eidx, ew)[:, :n, :]


def kernel(input, edge_index, edge_weight, W, b):
    n = input.shape[0]
    e = edge_weight.shape[0]
    nw = _NC * _NS
    per_tile = -(-e // nw)
    k_real = -(-(-(-per_tile // _CHUNK)) // _NIDX) * _NIDX
    k_alloc = k_real + 3
    slots = k_alloc * _CHUNK

    def to_tiles(a):
        a = jnp.pad(a, (0, nw * per_tile - e)).reshape(nw, per_tile)
        a = jnp.pad(a, ((0, 0), (0, slots - per_tile)))
        return a.reshape(nw, k_alloc, _CHUNK)

    eidx = jnp.stack([to_tiles(edge_index[0]), to_tiles(edge_index[1])],
                     axis=2)
    ew = to_tiles(edge_weight)

    support = _matmul(input.astype(jnp.float32), W.astype(jnp.float32))
    partials = _sc_aggregate(support, eidx, ew, k_real)
    return _combine(partials, b)
